# baseline (speedup 1.0000x reference)
import functools

import jax
import jax.numpy as jnp
from jax import lax
from jax.experimental import pallas as pl
from jax.experimental.pallas import tpu as pltpu, tpu_sc as plsc

N_ROWS = 16384
N_COLS = 3328
N_SLICES = 26
SLICE_LEN = 128
WIN = SLICE_LEN + 8
ROWS_PER_CHUNK = 256
NCHUNK = N_ROWS // ROWS_PER_CHUNK
TOTAL_ITEMS = N_SLICES * NCHUNK
NUM_WORKERS = 32
ITEMS_PER_WORKER = TOTAL_ITEMS // NUM_WORKERS
IDX_PAD = 48


def _slice_body(inp_hbm, idx_hbm, out_hbm, idx_v, win_v, out_v):
    wid = lax.axis_index("s") * 2 + lax.axis_index("c")
    pltpu.sync_copy(idx_hbm, idx_v)

    def item(t, carry):
        it = wid * ITEMS_PER_WORKER + t
        i = it // NCHUNK
        c = it - i * NCHUNK
        s = idx_v[pl.ds(i, 16)][0]
        col0 = pl.multiple_of(s & ~7, 8)
        off = s & 7
        r0 = c * ROWS_PER_CHUNK

        pltpu.sync_copy(
            inp_hbm.at[pl.ds(r0, ROWS_PER_CHUNK), pl.ds(col0, WIN)], win_v
        )

        def shift_row(r, carry2):
            for j in range(SLICE_LEN // 16):
                out_v[r, pl.ds(16 * j, 16)] = win_v[r, pl.ds(off + 16 * j, 16)]
            return carry2

        lax.fori_loop(0, ROWS_PER_CHUNK, shift_row, 0)

        pltpu.sync_copy(out_v, out_hbm.at[i, pl.ds(r0, ROWS_PER_CHUNK), :])
        return carry

    lax.fori_loop(0, ITEMS_PER_WORKER, item, 0)


def kernel(input_tensor, slices_index, slice_len):
    idx_padded = jnp.zeros((IDX_PAD,), jnp.int32).at[:N_SLICES].set(slices_index)
    mesh = plsc.VectorSubcoreMesh(core_axis_name="c", subcore_axis_name="s")
    run = pl.kernel(
        _slice_body,
        out_type=jax.ShapeDtypeStruct((N_SLICES, N_ROWS, SLICE_LEN), jnp.float32),
        mesh=mesh,
        scratch_types=[
            pltpu.VMEM((IDX_PAD,), jnp.int32),
            pltpu.VMEM((ROWS_PER_CHUNK, WIN), jnp.float32),
            pltpu.VMEM((ROWS_PER_CHUNK, SLICE_LEN), jnp.float32),
        ],
        compiler_params=pltpu.CompilerParams(use_tc_tiling_on_sc=False),
    )
    return run(input_tensor, idx_padded)


# speedup vs baseline: 1.2727x; 1.2727x over previous
import functools

import jax
import jax.numpy as jnp
from jax import lax
from jax.experimental import pallas as pl
from jax.experimental.pallas import tpu as pltpu, tpu_sc as plsc

N_ROWS = 16384
N_COLS = 3328
N_SLICES = 26
SLICE_LEN = 128
WIN = SLICE_LEN + 8
ROWS_PER_CHUNK = 128
NCHUNK = N_ROWS // ROWS_PER_CHUNK
TOTAL_ITEMS = N_SLICES * NCHUNK
NUM_WORKERS = 32
ITEMS_PER_WORKER = TOTAL_ITEMS // NUM_WORKERS
IDX_PAD = 48


def _decode(idx_v, wid, t):
    it = wid * ITEMS_PER_WORKER + t
    i = it // NCHUNK
    c = it - i * NCHUNK
    s = idx_v[pl.ds(i, 16)][0]
    col0 = pl.multiple_of(s & ~7, 8)
    off = s & 7
    r0 = c * ROWS_PER_CHUNK
    return i, r0, col0, off


def _slice_body(inp_hbm, idx_hbm, out_hbm, idx_v, win_v, out_v, gsem, ssem):
    wid = lax.axis_index("s") * 2 + lax.axis_index("c")
    pltpu.sync_copy(idx_hbm, idx_v)

    def start_gather(b, t):
        _, r0, col0, _ = _decode(idx_v, wid, t)
        pltpu.async_copy(
            inp_hbm.at[pl.ds(r0, ROWS_PER_CHUNK), pl.ds(col0, WIN)],
            win_v.at[b],
            gsem.at[b],
        )

    start_gather(0, 0)
    start_gather(1, 1)

    def pair(k, carry):
        for b in range(2):
            tt = 2 * k + b
            i, r0, col0, off = _decode(idx_v, wid, tt)
            pltpu.make_async_copy(
                inp_hbm.at[pl.ds(r0, ROWS_PER_CHUNK), pl.ds(col0, WIN)],
                win_v.at[b],
                gsem.at[b],
            ).wait()
            @pl.when(tt >= 2)
            def _():
                pltpu.make_async_copy(
                    out_v.at[b],
                    out_hbm.at[0, pl.ds(0, ROWS_PER_CHUNK), :],
                    ssem.at[b],
                ).wait()

            def shift_row(r, carry2):
                for j in range(SLICE_LEN // 16):
                    out_v[b, r, pl.ds(16 * j, 16)] = win_v[
                        b, r, pl.ds(off + 16 * j, 16)
                    ]
                return carry2

            lax.fori_loop(0, ROWS_PER_CHUNK, shift_row, 0)

            pltpu.async_copy(
                out_v.at[b],
                out_hbm.at[i, pl.ds(r0, ROWS_PER_CHUNK), :],
                ssem.at[b],
            )

            @pl.when(tt + 2 < ITEMS_PER_WORKER)
            def _():
                start_gather(b, tt + 2)

        return carry

    lax.fori_loop(0, ITEMS_PER_WORKER // 2, pair, 0)

    for b in range(2):
        pltpu.make_async_copy(
            out_v.at[b],
            out_hbm.at[0, pl.ds(0, ROWS_PER_CHUNK), :],
            ssem.at[b],
        ).wait()


def kernel(input_tensor, slices_index, slice_len):
    idx_padded = jnp.zeros((IDX_PAD,), jnp.int32).at[:N_SLICES].set(slices_index)
    mesh = plsc.VectorSubcoreMesh(core_axis_name="c", subcore_axis_name="s")
    run = pl.kernel(
        _slice_body,
        out_type=jax.ShapeDtypeStruct((N_SLICES, N_ROWS, SLICE_LEN), jnp.float32),
        mesh=mesh,
        scratch_types=[
            pltpu.VMEM((IDX_PAD,), jnp.int32),
            pltpu.VMEM((2, ROWS_PER_CHUNK, WIN), jnp.float32),
            pltpu.VMEM((2, ROWS_PER_CHUNK, SLICE_LEN), jnp.float32),
            pltpu.SemaphoreType.DMA((2,)),
            pltpu.SemaphoreType.DMA((2,)),
        ],
        compiler_params=pltpu.CompilerParams(use_tc_tiling_on_sc=False),
    )
    return run(input_tensor, idx_padded)


# speedup vs baseline: 1.4558x; 1.1439x over previous
import functools

import jax
import jax.numpy as jnp
from jax import lax
from jax.experimental import pallas as pl
from jax.experimental.pallas import tpu as pltpu, tpu_sc as plsc

N_ROWS = 16384
N_COLS = 3328
N_SLICES = 26
SLICE_LEN = 128
WIN = 2 * SLICE_LEN
ROWS_PER_CHUNK = 128
NCHUNK = N_ROWS // ROWS_PER_CHUNK
TOTAL_ITEMS = N_SLICES * NCHUNK
NUM_WORKERS = 32
ITEMS_PER_WORKER = TOTAL_ITEMS // NUM_WORKERS
IDX_PAD = 48


def _decode(idx_v, wid, t):
    it = wid * ITEMS_PER_WORKER + t
    i = it // NCHUNK
    c = it - i * NCHUNK
    s = idx_v[pl.ds(i, 16)][0]
    col0 = pl.multiple_of(s & ~127, 128)
    off16 = s & 112
    rem = s & 15
    r0 = c * ROWS_PER_CHUNK
    return i, r0, col0, off16, rem


def _slice_body(inp_hbm, idx_hbm, out_hbm, idx_v, win_v, out_v, gsem, ssem):
    wid = lax.axis_index("s") * 2 + lax.axis_index("c")
    pltpu.sync_copy(idx_hbm, idx_v)

    def start_gather(b, t):
        _, r0, col0, _, _ = _decode(idx_v, wid, t)
        pltpu.async_copy(
            inp_hbm.at[pl.ds(r0, ROWS_PER_CHUNK), pl.ds(col0, WIN)],
            win_v.at[b],
            gsem.at[b],
        )

    start_gather(0, 0)
    start_gather(1, 1)

    def pair(k, carry):
        for b in range(2):
            tt = 2 * k + b
            i, r0, col0, off16, rem = _decode(idx_v, wid, tt)
            pltpu.make_async_copy(
                inp_hbm.at[pl.ds(r0, ROWS_PER_CHUNK), pl.ds(col0, WIN)],
                win_v.at[b],
                gsem.at[b],
            ).wait()
            @pl.when(tt >= 2)
            def _():
                pltpu.make_async_copy(
                    out_v.at[b],
                    out_hbm.at[0, pl.ds(0, ROWS_PER_CHUNK), :],
                    ssem.at[b],
                ).wait()

            lane = lax.iota(jnp.int32, 16)
            ia = lane + rem
            ia_lo = ia & 15
            from_a = ia < 16

            def pick(vec, idx):
                return lax.gather(
                    vec,
                    idx[:, None],
                    lax.GatherDimensionNumbers(
                        offset_dims=(),
                        collapsed_slice_dims=(0,),
                        start_index_map=(0,),
                    ),
                    (1,),
                    mode=lax.GatherScatterMode.PROMISE_IN_BOUNDS,
                )

            def shift_row(r, carry2):
                a = win_v[b, r, pl.ds(pl.multiple_of(off16, 16), 16)]
                for j in range(SLICE_LEN // 16):
                    bvec = win_v[
                        b, r, pl.ds(pl.multiple_of(off16 + 16 * j + 16, 16), 16)
                    ]
                    va = pick(a, ia_lo)
                    vb = pick(bvec, ia_lo)
                    out_v[b, r, pl.ds(16 * j, 16)] = jnp.where(from_a, va, vb)
                    a = bvec
                return carry2

            lax.fori_loop(0, ROWS_PER_CHUNK, shift_row, 0)

            pltpu.async_copy(
                out_v.at[b],
                out_hbm.at[i, pl.ds(r0, ROWS_PER_CHUNK), :],
                ssem.at[b],
            )

            @pl.when(tt + 2 < ITEMS_PER_WORKER)
            def _():
                start_gather(b, tt + 2)

        return carry

    lax.fori_loop(0, ITEMS_PER_WORKER // 2, pair, 0)

    for b in range(2):
        pltpu.make_async_copy(
            out_v.at[b],
            out_hbm.at[0, pl.ds(0, ROWS_PER_CHUNK), :],
            ssem.at[b],
        ).wait()


def kernel(input_tensor, slices_index, slice_len):
    idx_padded = jnp.zeros((IDX_PAD,), jnp.int32).at[:N_SLICES].set(slices_index)
    mesh = plsc.VectorSubcoreMesh(core_axis_name="c", subcore_axis_name="s")
    run = pl.kernel(
        _slice_body,
        out_type=jax.ShapeDtypeStruct((N_SLICES, N_ROWS, SLICE_LEN), jnp.float32),
        mesh=mesh,
        scratch_types=[
            pltpu.VMEM((IDX_PAD,), jnp.int32),
            pltpu.VMEM((2, ROWS_PER_CHUNK, WIN), jnp.float32),
            pltpu.VMEM((2, ROWS_PER_CHUNK, SLICE_LEN), jnp.float32),
            pltpu.SemaphoreType.DMA((2,)),
            pltpu.SemaphoreType.DMA((2,)),
        ],
    )
    return run(input_tensor, idx_padded)
